# baseline (speedup 1.0000x reference)
import functools

import jax
import jax.numpy as jnp
from jax.experimental import pallas as pl

BUFFER_SIZE = 2000000
RESOLUTION = 0.3
NUM_BANDS = 64
FEATURE_DIM = 32
N_POINTS = 262144

_PE_BLOCK = 2048


def _pe_body(points_ref, bpe_ref, gathered_ref, out_ref):
    pts = points_ref[...]
    bpe = bpe_ref[...]
    px = pts[:, 0:1]
    py = pts[:, 1:2]
    pz = pts[:, 2:3]
    two_pi = 2.0 * jnp.pi
    bf = lambda a: a.astype(jnp.bfloat16).astype(jnp.float32)
    xp = (bf(px) * bf(bpe[0:1, :]) + bf(py) * bf(bpe[1:2, :])
          + bf(pz) * bf(bpe[2:3, :])) * two_pi
    c1 = jnp.float32(6.28125)
    c2 = jnp.float32(0.0019350052)
    c3 = jnp.float32(3.0198134e-07)
    c4 = jnp.float32(1.0253132e-11)
    n = jnp.round(xp * jnp.float32(1.0 / two_pi))
    r = (((xp - n * c1) - n * c2) - n * c3) - n * c4
    out_ref[...] = jnp.concatenate(
        [pts, jnp.sin(r), jnp.cos(r), gathered_ref[...]], axis=1)


def _pe_concat(points, B_pe, gathered):
    n = points.shape[0]
    grid = (n // _PE_BLOCK,)
    return pl.pallas_call(
        _pe_body,
        grid=grid,
        in_specs=[
            pl.BlockSpec((_PE_BLOCK, 3), lambda i: (i, i * 0)),
            pl.BlockSpec((3, NUM_BANDS), lambda i: (i * 0, i * 0)),
            pl.BlockSpec((_PE_BLOCK, FEATURE_DIM), lambda i: (i, i * 0)),
        ],
        out_specs=pl.BlockSpec((_PE_BLOCK, 3 + 2 * NUM_BANDS + FEATURE_DIM),
                               lambda i: (i, i * 0)),
        out_shape=jax.ShapeDtypeStruct(
            (n, 3 + 2 * NUM_BANDS + FEATURE_DIM), jnp.float32),
    )(points, B_pe, gathered)


def kernel(points, values, mem, B_pe):
    primes = jnp.array([73856093, 19349669, 83492791], dtype=jnp.int64)
    grid = jnp.floor(points / RESOLUTION).astype(jnp.int64)
    keys = ((grid * primes).sum(axis=-1) % BUFFER_SIZE).astype(jnp.int32)
    summed = jnp.zeros((BUFFER_SIZE, FEATURE_DIM), jnp.float32).at[keys].add(values)
    gathered = jnp.take(summed, keys, axis=0)
    return _pe_concat(points, B_pe, gathered)
